# Initial kernel scaffold; baseline (speedup 1.0000x reference)
import jax
import jax.numpy as jnp
from jax.experimental import pallas as pl


def kernel(atom, bond, bond_idx, W, b):
    raise NotImplementedError("write your pallas kernel here")


# speedup vs baseline: 1.9183x; 1.9183x over previous
import functools

import jax
import jax.numpy as jnp
from jax import lax
from jax.experimental import pallas as pl
from jax.experimental.pallas import tpu as pltpu
from jax.experimental.pallas import tpu_sc as plsc

N = 10000
M = 32
D = 128
C = 256
BD = 16
EPS = 1e-5

GR = 6400
GC = 50
NC = 2
NS = 16
NW = NC * NS
ROWS_W = GR // NW
SCH = 8

NB = 200
NBLK = N // NB
R = NB * M


def _sc_gather(atom, idx2d):
    mesh = plsc.VectorSubcoreMesh(core_axis_name="c", subcore_axis_name="s")

    @functools.partial(
        pl.kernel,
        mesh=mesh,
        out_type=jax.ShapeDtypeStruct((GR, GC, D), jnp.float32),
        scratch_types=[
            pltpu.VMEM((SCH, GC), jnp.int32),
            pltpu.VMEM((SCH, GC, D), jnp.float32),
            pltpu.SemaphoreType.DMA,
        ],
    )
    def gk(table_hbm, idx_hbm, out_hbm, idx_v, rows_v, sem):
        wid = lax.axis_index("s") * NC + lax.axis_index("c")

        def body(j, carry):
            r0 = wid * ROWS_W + j * SCH
            pltpu.sync_copy(idx_hbm.at[pl.ds(r0, SCH)], idx_v)
            cps = [
                pltpu.async_copy(table_hbm.at[idx_v.at[i]], rows_v.at[i], sem)
                for i in range(SCH)
            ]
            for cp in cps:
                cp.wait()
            pltpu.sync_copy(rows_v, out_hbm.at[pl.ds(r0, SCH)])
            return carry

        lax.fori_loop(0, ROWS_W // SCH, body, 0)

    return gk(atom, idx2d)


def _compute_x(g_ref, bond_ref, atom_ref, wnt_ref, wbt_ref, wct_ref, b_ref):
    xg = jnp.dot(g_ref[...], wnt_ref[...], preferred_element_type=jnp.float32)
    xb = jnp.dot(bond_ref[...], wbt_ref[...], preferred_element_type=jnp.float32)
    pc = jnp.dot(atom_ref[...], wct_ref[...], preferred_element_type=jnp.float32)
    pc = pc + b_ref[0:1, :]
    return (xg + xb).reshape(NB, M, C) + pc[:, None, :]


def _stats_body(g_ref, bond_ref, atom_ref, wnt_ref, wbt_ref, wct_ref, b_ref,
                s1_ref, s2_ref):
    i = pl.program_id(0)
    x = _compute_x(g_ref, bond_ref, atom_ref, wnt_ref, wbt_ref, wct_ref, b_ref)
    s1 = jnp.sum(x, axis=(0, 1), keepdims=True).reshape(1, C)
    s2 = jnp.sum(x * x, axis=(0, 1), keepdims=True).reshape(1, C)

    @pl.when(i == 0)
    def _():
        s1_ref[...] = jnp.zeros_like(s1_ref)
        s2_ref[...] = jnp.zeros_like(s2_ref)

    s1_ref[...] += jnp.broadcast_to(s1, (8, C))
    s2_ref[...] += jnp.broadcast_to(s2, (8, C))


def _softplus(x):
    return jnp.maximum(x, 0.0) + jnp.log1p(jnp.exp(-jnp.abs(x)))


def _apply_body(g_ref, bond_ref, atom_ref, wnt_ref, wbt_ref, wct_ref, b_ref,
                sc_ref, sh_ref, s_ref, t1_ref, t2_ref):
    i = pl.program_id(0)
    x = _compute_x(g_ref, bond_ref, atom_ref, wnt_ref, wbt_ref, wct_ref, b_ref)
    xn = x * sc_ref[0:1, :].reshape(1, 1, C) + sh_ref[0:1, :].reshape(1, 1, C)
    filt = 1.0 / (1.0 + jnp.exp(-xn[:, :, :D]))
    core = _softplus(xn[:, :, D:])
    s = jnp.sum(filt * core, axis=1)
    s_ref[...] = s
    t1 = jnp.sum(s, axis=0, keepdims=True)
    t2 = jnp.sum(s * s, axis=0, keepdims=True)

    @pl.when(i == 0)
    def _():
        t1_ref[...] = jnp.zeros_like(t1_ref)
        t2_ref[...] = jnp.zeros_like(t2_ref)

    t1_ref[...] += jnp.broadcast_to(t1, (8, D))
    t2_ref[...] += jnp.broadcast_to(t2, (8, D))


def _final_body(atom_ref, s_ref, sc2_ref, sh2_ref, out_ref):
    sn = s_ref[...] * sc2_ref[0:1, :] + sh2_ref[0:1, :]
    out_ref[...] = _softplus(atom_ref[...] + sn)


def kernel(atom, bond, bond_idx, W, b):
    wct = W[:, :D].T
    wnt = W[:, D:2 * D].T
    wbt = W[:, 2 * D:].T
    b8 = jnp.broadcast_to(b.reshape(1, C), (8, C))

    idx2d = bond_idx.reshape(GR, GC)
    g3 = _sc_gather(atom, idx2d)
    g2 = g3.reshape(N * M, D)
    bond2 = bond.reshape(N * M, BD)

    full = lambda shape: pl.BlockSpec(shape, lambda i: (0, 0))
    row_specs = [
        pl.BlockSpec((R, D), lambda i: (i, 0)),
        pl.BlockSpec((R, BD), lambda i: (i, 0)),
        pl.BlockSpec((NB, D), lambda i: (i, 0)),
        full((D, C)), full((BD, C)), full((D, C)), full((8, C)),
    ]

    s1, s2 = pl.pallas_call(
        _stats_body,
        grid=(NBLK,),
        in_specs=row_specs,
        out_specs=[full((8, C)), full((8, C))],
        out_shape=[jax.ShapeDtypeStruct((8, C), jnp.float32)] * 2,
    )(g2, bond2, atom, wnt, wbt, wct, b8)

    cnt = float(N * M)
    mean = s1[0] / cnt
    var = s2[0] / cnt - mean * mean
    scale = lax.rsqrt(var + EPS)
    shift = -mean * scale
    sc8 = jnp.broadcast_to(scale.reshape(1, C), (8, C))
    sh8 = jnp.broadcast_to(shift.reshape(1, C), (8, C))

    s, t1, t2 = pl.pallas_call(
        _apply_body,
        grid=(NBLK,),
        in_specs=row_specs + [full((8, C)), full((8, C))],
        out_specs=[pl.BlockSpec((NB, D), lambda i: (i, 0)),
                   full((8, D)), full((8, D))],
        out_shape=[jax.ShapeDtypeStruct((N, D), jnp.float32),
                   jax.ShapeDtypeStruct((8, D), jnp.float32),
                   jax.ShapeDtypeStruct((8, D), jnp.float32)],
    )(g2, bond2, atom, wnt, wbt, wct, b8, sc8, sh8)

    mean2 = t1[0] / float(N)
    var2 = t2[0] / float(N) - mean2 * mean2
    scale2 = lax.rsqrt(var2 + EPS)
    shift2 = -mean2 * scale2
    sc28 = jnp.broadcast_to(scale2.reshape(1, D), (8, D))
    sh28 = jnp.broadcast_to(shift2.reshape(1, D), (8, D))

    out = pl.pallas_call(
        _final_body,
        grid=(NBLK,),
        in_specs=[pl.BlockSpec((NB, D), lambda i: (i, 0)),
                  pl.BlockSpec((NB, D), lambda i: (i, 0)),
                  full((8, D)), full((8, D))],
        out_specs=pl.BlockSpec((NB, D), lambda i: (i, 0)),
        out_shape=jax.ShapeDtypeStruct((N, D), jnp.float32),
    )(atom, s, sc28, sh28)
    return out


# speedup vs baseline: 2.1116x; 1.1007x over previous
import functools

import jax
import jax.numpy as jnp
from jax import lax
from jax.experimental import pallas as pl
from jax.experimental.pallas import tpu as pltpu
from jax.experimental.pallas import tpu_sc as plsc

N = 10000
M = 32
D = 128
C = 256
BD = 16
EPS = 1e-5

NC = 2
NS = 16
NW = NC * NS
CHROWS = 400
GSZ = 80
GPC = CHROWS // GSZ
NCHUNK = (N * M) // (NW * CHROWS)

NB = 200
NBLK = N // NB
R = NB * M


DW = D // 2


def _sc_gather(table, idx1d):
    mesh = plsc.VectorSubcoreMesh(core_axis_name="c", subcore_axis_name="s")

    @functools.partial(
        pl.kernel,
        mesh=mesh,
        out_type=jax.ShapeDtypeStruct((N * M, DW), jnp.int32),
        scratch_types=[
            pltpu.VMEM((CHROWS,), jnp.int32),
            pltpu.VMEM((CHROWS, DW), jnp.int32),
            pltpu.SemaphoreType.DMA,
        ],
        compiler_params=pltpu.CompilerParams(use_tc_tiling_on_sc=False),
    )
    def gk(table_hbm, idx_hbm, out_hbm, idx_v, rows_v, sem):
        wid = lax.axis_index("s") * NC + lax.axis_index("c")

        def body(j, carry):
            r0 = (wid * NCHUNK + j) * CHROWS
            pltpu.sync_copy(idx_hbm.at[pl.ds(r0, CHROWS)], idx_v)
            cps = [
                pltpu.async_copy(table_hbm.at[idx_v.at[pl.ds(g * GSZ, GSZ)]],
                                 rows_v.at[pl.ds(g * GSZ, GSZ)], sem)
                for g in range(GPC)
            ]
            for cp in cps:
                cp.wait()
            pltpu.sync_copy(rows_v, out_hbm.at[pl.ds(r0, CHROWS)])
            return carry

        lax.fori_loop(0, NCHUNK, body, 0)

    return gk(table, idx1d)


def _compute_x(g_ref, bond_ref, atom_ref, wnt_ref, wbt_ref, wct_ref, b_ref):
    g = g_ref[...]
    g_lo = lax.bitcast_convert_type(g << 16, jnp.float32).astype(jnp.bfloat16)
    g_hi = lax.bitcast_convert_type(g & jnp.int32(-65536),
                                    jnp.float32).astype(jnp.bfloat16)
    gcat = jnp.concatenate([g_lo, g_hi], axis=1)
    xg = jnp.dot(gcat, wnt_ref[...], preferred_element_type=jnp.float32)
    xb = jnp.dot(bond_ref[...].reshape(R, BD).astype(jnp.bfloat16),
                 wbt_ref[...], preferred_element_type=jnp.float32)
    pc = jnp.dot(atom_ref[...].astype(jnp.bfloat16), wct_ref[...],
                 preferred_element_type=jnp.float32)
    pc = pc + b_ref[0:1, :]
    return (xg + xb).reshape(NB, M, C) + pc[:, None, :]


def _stats_body(g_ref, bond_ref, atom_ref, wnt_ref, wbt_ref, wct_ref, b_ref,
                s1_ref, s2_ref):
    i = pl.program_id(0)
    x = _compute_x(g_ref, bond_ref, atom_ref, wnt_ref, wbt_ref, wct_ref, b_ref)
    s1 = jnp.sum(x, axis=(0, 1), keepdims=True).reshape(1, C)
    s2 = jnp.sum(x * x, axis=(0, 1), keepdims=True).reshape(1, C)

    @pl.when(i == 0)
    def _():
        s1_ref[...] = jnp.zeros_like(s1_ref)
        s2_ref[...] = jnp.zeros_like(s2_ref)

    s1_ref[...] += jnp.broadcast_to(s1, (8, C))
    s2_ref[...] += jnp.broadcast_to(s2, (8, C))


def _softplus(x):
    return jnp.maximum(x, 0.0) + jnp.log1p(jnp.exp(-jnp.abs(x)))


def _apply_body(g_ref, bond_ref, atom_ref, wnt_ref, wbt_ref, wct_ref, b_ref,
                sel_ref, s_ref, t1_ref, t2_ref):
    i = pl.program_id(0)
    xn = _compute_x(g_ref, bond_ref, atom_ref, wnt_ref, wbt_ref, wct_ref,
                    b_ref).reshape(R, C)
    filt = 1.0 / (1.0 + jnp.exp(-xn[:, :D]))
    core = _softplus(xn[:, D:])
    prod = (filt * core).astype(jnp.bfloat16)
    s = jnp.dot(sel_ref[...], prod, preferred_element_type=jnp.float32)
    s_ref[...] = s
    t1 = jnp.sum(s, axis=0, keepdims=True)
    t2 = jnp.sum(s * s, axis=0, keepdims=True)

    @pl.when(i == 0)
    def _():
        t1_ref[...] = jnp.zeros_like(t1_ref)
        t2_ref[...] = jnp.zeros_like(t2_ref)

    t1_ref[...] += jnp.broadcast_to(t1, (8, D))
    t2_ref[...] += jnp.broadcast_to(t2, (8, D))


def _final_body(atom_ref, s_ref, sc2_ref, sh2_ref, out_ref):
    sn = s_ref[...] * sc2_ref[0:1, :] + sh2_ref[0:1, :]
    out_ref[...] = _softplus(atom_ref[...] + sn)


def kernel(atom, bond, bond_idx, W, b):
    wct = W[:, :D].T.astype(jnp.bfloat16)
    wnt_n = W[:, D:2 * D].T.astype(jnp.bfloat16)
    wnt = jnp.concatenate([wnt_n[0::2], wnt_n[1::2]], axis=0)
    wbt = W[:, 2 * D:].T.astype(jnp.bfloat16)
    b8 = jnp.broadcast_to(b.reshape(1, C), (8, C))

    idx1d = bond_idx.reshape(N * M)
    atom_packed = lax.bitcast_convert_type(
        atom.astype(jnp.bfloat16).reshape(N, DW, 2), jnp.int32)
    g2 = _sc_gather(atom_packed, idx1d)

    full = lambda shape: pl.BlockSpec(shape, lambda i: (0, 0))
    row_specs = [
        pl.BlockSpec((R, DW), lambda i: (i, 0)),
        pl.BlockSpec((NB, M, BD), lambda i: (i, 0, 0)),
        pl.BlockSpec((NB, D), lambda i: (i, 0)),
        full((D, C)), full((BD, C)), full((D, C)), full((8, C)),
    ]

    s1, s2 = pl.pallas_call(
        _stats_body,
        grid=(NBLK,),
        in_specs=row_specs,
        out_specs=[full((8, C)), full((8, C))],
        out_shape=[jax.ShapeDtypeStruct((8, C), jnp.float32)] * 2,
    )(g2, bond, atom, wnt, wbt, wct, b8)

    cnt = float(N * M)
    mean = s1[0] / cnt
    var = s2[0] / cnt - mean * mean
    scale = lax.rsqrt(var + EPS)
    shift = -mean * scale
    wnt_s = (wnt.astype(jnp.float32) * scale).astype(jnp.bfloat16)
    wbt_s = (wbt.astype(jnp.float32) * scale).astype(jnp.bfloat16)
    wct_s = (wct.astype(jnp.float32) * scale).astype(jnp.bfloat16)
    bsh8 = jnp.broadcast_to((b * scale + shift).reshape(1, C), (8, C))
    sel = (lax.broadcasted_iota(jnp.int32, (NB, R), 1) // M
           == lax.broadcasted_iota(jnp.int32, (NB, R), 0)).astype(jnp.bfloat16)

    apply_specs = [
        pl.BlockSpec((R, DW), lambda i: (i, 0)),
        pl.BlockSpec((NB, M, BD), lambda i: (i, 0, 0)),
        pl.BlockSpec((NB, D), lambda i: (i, 0)),
        full((D, C)), full((BD, C)), full((D, C)), full((8, C)),
        full((NB, R)),
    ]
    s, t1, t2 = pl.pallas_call(
        _apply_body,
        grid=(NBLK,),
        in_specs=apply_specs,
        out_specs=[pl.BlockSpec((NB, D), lambda i: (i, 0)),
                   full((8, D)), full((8, D))],
        out_shape=[jax.ShapeDtypeStruct((N, D), jnp.float32),
                   jax.ShapeDtypeStruct((8, D), jnp.float32),
                   jax.ShapeDtypeStruct((8, D), jnp.float32)],
    )(g2, bond, atom, wnt_s, wbt_s, wct_s, bsh8, sel)

    mean2 = t1[0] / float(N)
    var2 = t2[0] / float(N) - mean2 * mean2
    scale2 = lax.rsqrt(var2 + EPS)
    shift2 = -mean2 * scale2
    sc28 = jnp.broadcast_to(scale2.reshape(1, D), (8, D))
    sh28 = jnp.broadcast_to(shift2.reshape(1, D), (8, D))

    out = pl.pallas_call(
        _final_body,
        grid=(NBLK,),
        in_specs=[pl.BlockSpec((NB, D), lambda i: (i, 0)),
                  pl.BlockSpec((NB, D), lambda i: (i, 0)),
                  full((8, D)), full((8, D))],
        out_specs=pl.BlockSpec((NB, D), lambda i: (i, 0)),
        out_shape=jax.ShapeDtypeStruct((N, D), jnp.float32),
    )(atom, s, sc28, sh28)
    return out
